# baseline (speedup 1.0000x reference)
import jax
import jax.numpy as jnp
from jax import lax
from jax.experimental import pallas as pl
from jax.experimental.pallas import tpu as pltpu

_K = 16


def _coeffs(t_ref, ab_ref, b):
    a = ab_ref[t_ref[b]]
    return jax.lax.rsqrt(a), jnp.sqrt(1.0 / a - 1.0)


def _body(t_ref, ab_ref, x_hbm, n_hbm, o_hbm, xb, nb, ob, sx, sn, so):
    nch = pl.num_programs(0)
    i = pl.program_id(0)
    slot = lax.rem(i, _K)
    HH = xb.shape[2]

    def src(ref, k):
        return ref.at[lax.div(k, 2), :, pl.ds(lax.rem(k, 2) * HH, HH), :]

    @pl.when(i == 0)
    def _prologue():
        for j in range(_K):
            pltpu.make_async_copy(src(x_hbm, j), xb.at[j], sx.at[j]).start()
            pltpu.make_async_copy(src(n_hbm, j), nb.at[j], sn.at[j]).start()

    pltpu.make_async_copy(src(x_hbm, i), xb.at[slot], sx.at[slot]).wait()
    pltpu.make_async_copy(src(n_hbm, i), nb.at[slot], sn.at[slot]).wait()

    @pl.when(i >= _K)
    def _drain_out():
        pltpu.make_async_copy(ob.at[slot], src(o_hbm, i - _K), so.at[slot]).wait()

    c1, c2 = _coeffs(t_ref, ab_ref, lax.div(i, 2))
    ob.at[slot][...] = c1 * xb.at[slot][...] - c2 * nb.at[slot][...]
    pltpu.make_async_copy(ob.at[slot], src(o_hbm, i), so.at[slot]).start()

    @pl.when(i + _K < nch)
    def _prefetch():
        pltpu.make_async_copy(src(x_hbm, i + _K), xb.at[slot], sx.at[slot]).start()
        pltpu.make_async_copy(src(n_hbm, i + _K), nb.at[slot], sn.at[slot]).start()

    @pl.when(i == nch - 1)
    def _epilogue():
        for j in range(_K):
            pltpu.make_async_copy(
                ob.at[j], src(o_hbm, nch - _K + j), so.at[j]
            ).wait()


def kernel(x_t, t, pred_noise, alphas_bar):
    B, C, H, W = x_t.shape
    HH = H // 2

    out = pl.pallas_call(
        _body,
        grid=(2 * B,),
        in_specs=[
            pl.BlockSpec(memory_space=pltpu.SMEM),
            pl.BlockSpec(memory_space=pltpu.SMEM),
            pl.BlockSpec(memory_space=pltpu.MemorySpace.HBM),
            pl.BlockSpec(memory_space=pltpu.MemorySpace.HBM),
        ],
        out_specs=pl.BlockSpec(memory_space=pltpu.MemorySpace.HBM),
        out_shape=jax.ShapeDtypeStruct((B, C, H, W), jnp.float32),
        scratch_shapes=[
            pltpu.VMEM((_K, C, HH, W), jnp.float32),
            pltpu.VMEM((_K, C, HH, W), jnp.float32),
            pltpu.VMEM((_K, C, HH, W), jnp.float32),
            pltpu.SemaphoreType.DMA((_K,)),
            pltpu.SemaphoreType.DMA((_K,)),
            pltpu.SemaphoreType.DMA((_K,)),
        ],
    )(t, alphas_bar, x_t, pred_noise)

    return out


# speedup vs baseline: 1.0002x; 1.0002x over previous
import jax
import jax.numpy as jnp
from jax import lax
from jax.experimental import pallas as pl
from jax.experimental.pallas import tpu as pltpu

_K = 8


def _coeffs(t_ref, ab_ref, i):
    a = ab_ref[t_ref[i]]
    return jax.lax.rsqrt(a), jnp.sqrt(1.0 / a - 1.0)


def _body(t_ref, ab_ref, x_hbm, n_hbm, o_hbm, xb, nb, ob, sx, sn, so):
    nch = pl.num_programs(0)
    i = pl.program_id(0)
    slot = lax.rem(i, _K)

    @pl.when(i == 0)
    def _prologue():
        for j in range(_K):
            pltpu.make_async_copy(x_hbm.at[j], xb.at[j], sx.at[j]).start()
            pltpu.make_async_copy(n_hbm.at[j], nb.at[j], sn.at[j]).start()

    pltpu.make_async_copy(x_hbm.at[i], xb.at[slot], sx.at[slot]).wait()
    pltpu.make_async_copy(n_hbm.at[i], nb.at[slot], sn.at[slot]).wait()

    @pl.when(i >= _K)
    def _drain_out():
        pltpu.make_async_copy(ob.at[slot], o_hbm.at[i - _K], so.at[slot]).wait()

    c1, c2 = _coeffs(t_ref, ab_ref, i)
    ob.at[slot][...] = c1 * xb.at[slot][...] - c2 * nb.at[slot][...]
    pltpu.make_async_copy(ob.at[slot], o_hbm.at[i], so.at[slot]).start()

    @pl.when(i + _K < nch)
    def _prefetch():
        pltpu.make_async_copy(x_hbm.at[i + _K], xb.at[slot], sx.at[slot]).start()
        pltpu.make_async_copy(n_hbm.at[i + _K], nb.at[slot], sn.at[slot]).start()

    @pl.when(i == nch - 1)
    def _epilogue():
        for j in range(_K):
            pltpu.make_async_copy(
                ob.at[j], o_hbm.at[nch - _K + j], so.at[j]
            ).wait()


def kernel(x_t, t, pred_noise, alphas_bar):
    B, C, H, W = x_t.shape

    out = pl.pallas_call(
        _body,
        grid=(B,),
        in_specs=[
            pl.BlockSpec(memory_space=pltpu.SMEM),
            pl.BlockSpec(memory_space=pltpu.SMEM),
            pl.BlockSpec(memory_space=pltpu.MemorySpace.HBM),
            pl.BlockSpec(memory_space=pltpu.MemorySpace.HBM),
        ],
        out_specs=pl.BlockSpec(memory_space=pltpu.MemorySpace.HBM),
        out_shape=jax.ShapeDtypeStruct((B, C, H, W), jnp.float32),
        scratch_shapes=[
            pltpu.VMEM((_K, C, H, W), jnp.float32),
            pltpu.VMEM((_K, C, H, W), jnp.float32),
            pltpu.VMEM((_K, C, H, W), jnp.float32),
            pltpu.SemaphoreType.DMA((_K,)),
            pltpu.SemaphoreType.DMA((_K,)),
            pltpu.SemaphoreType.DMA((_K,)),
        ],
    )(t, alphas_bar, x_t, pred_noise)

    return out
